# baseline (speedup 1.0000x reference)
import functools

import jax
import jax.numpy as jnp
from jax import lax
from jax.experimental import pallas as pl
from jax.experimental.pallas import tpu as pltpu
from jax.experimental.pallas import tpu_sc as plsc

N_NODES = 10000
N_EDGES = 320000
DIM = 128
NMED = 200
MOL_SIZE = 50
EPS = 0.3

NC = 2
NS = 16
NW = NC * NS
EPW = N_EDGES // NW
K = 80
NSUP = 25
SUPC = 5
NCHUNK = NSUP * SUPC
NGROUP = N_NODES // 16

_mesh = plsc.VectorSubcoreMesh(
    core_axis_name="c", subcore_axis_name="s", num_cores=NC, num_subcores=NS)

_f32 = jnp.float32
_i32 = jnp.int32


@functools.partial(
    pl.kernel,
    out_type=(
        jax.ShapeDtypeStruct((N_NODES, DIM), _f32),
        jax.ShapeDtypeStruct((NW * N_NODES,), _f32),
    ),
    mesh=_mesh,
    scratch_types=[
        pltpu.VMEM((16,), _i32),
        pltpu.VMEM((N_NODES,), _i32),
        pltpu.VMEM((N_NODES,), _f32),
        pltpu.VMEM((16, DIM), _f32),
        pltpu.SemaphoreType.DMA,
    ],
    compiler_params=pltpu.CompilerParams(needs_layout_passes=False),
)
def _sc_prep(fp_hbm, dst_hbm, tbl_hbm, h0_hbm, degp_hbm,
             fp_v, dstbuf_v, hist_v, rows_v, sem):
    c = lax.axis_index("c")
    s = lax.axis_index("s")
    wid = s * NC + c

    n_groups = jnp.where(wid < (NGROUP % NW) + ((NGROUP % NW == 0) * NW),
                         NGROUP // NW + 1, NGROUP // NW)

    def emb_body(i, carry):
        g = wid + NW * i
        pltpu.sync_copy(fp_hbm.at[pl.ds(g * 16, 16)], fp_v)
        pltpu.async_copy(tbl_hbm.at[fp_v], rows_v, sem).wait()
        pltpu.sync_copy(rows_v, h0_hbm.at[pl.ds(g * 16, 16)])
        return carry

    lax.fori_loop(0, n_groups, emb_body, 0)

    def zero_body(i, carry):
        hist_v[pl.ds(i * 16, 16)] = jnp.zeros((16,), _f32)
        return carry

    lax.fori_loop(0, NGROUP, zero_body, 0)
    pltpu.sync_copy(dst_hbm.at[pl.ds(wid * EPW, EPW)], dstbuf_v)
    ones = jnp.ones((16,), _f32)

    def hist_body(i, carry):
        dv = dstbuf_v[pl.ds(i * 16, 16)]
        plsc.addupdate_scatter(hist_v, [dv], ones)
        return carry

    lax.fori_loop(0, EPW // 16, hist_body, 0)
    pltpu.sync_copy(hist_v, degp_hbm.at[pl.ds(wid * N_NODES, N_NODES)])


@functools.partial(
    pl.kernel,
    out_type=jax.ShapeDtypeStruct((NC, N_NODES, DIM), _f32),
    mesh=_mesh,
    scratch_types=[
        pltpu.VMEM((N_NODES,), _f32),
        pltpu.VMEM((N_NODES,), _f32),
        pltpu.VMEM((2 * SUPC, K), _i32),
        pltpu.VMEM((2 * SUPC, K), _i32),
        pltpu.VMEM((K,), _f32),
        pltpu.VMEM((2 * K, DIM), _f32),
        pltpu.VMEM_SHARED((N_NODES, DIM), _f32),
        pltpu.SemaphoreType.DMA((2,)),
        pltpu.SemaphoreType.DMA((2,)),
    ],
    compiler_params=pltpu.CompilerParams(needs_layout_passes=False),
)
def _sc_edge(src4_hbm, dst4_hbm, hn_hbm, aa_hbm, bb_hbm, mparts_hbm,
             aa_v, bb_v, src_sup, dst_sup, w_v, rows2, macc, semg, sems):
    c = lax.axis_index("c")
    s = lax.axis_index("s")
    wid = s * NC + c

    for i in range(8):
        for j in range(DIM // 16):
            rows2[i, pl.ds(j * 16, 16)] = jnp.zeros((16,), _f32)
    ngrp8 = N_NODES // 8
    my_ng = jnp.where(s < ngrp8 % NS, ngrp8 // NS + 1, ngrp8 // NS)

    def zstripe(i, carry):
        g = s + NS * i
        pltpu.sync_copy(rows2.at[pl.ds(0, 8)], macc.at[pl.ds(g * 8, 8)])
        return carry

    lax.fori_loop(0, my_ng, zstripe, 0)

    pltpu.sync_copy(aa_hbm, aa_v)
    pltpu.sync_copy(bb_hbm, bb_v)
    pltpu.sync_copy(src4_hbm.at[wid, 0], src_sup.at[pl.ds(0, SUPC)])
    pltpu.sync_copy(dst4_hbm.at[wid, 0], dst_sup.at[pl.ds(0, SUPC)])
    plsc.subcore_barrier()
    pltpu.async_copy(hn_hbm.at[src_sup.at[0]], rows2.at[pl.ds(0, K)], semg.at[0])

    zero = jnp.int32(0)

    def chunk_body(ci, carry):
        p, su, r, pp = carry
        irow = pp * SUPC + r
        last_r = r == SUPC - 1
        q = 1 - p
        su2 = jnp.where(last_r, su + 1, su)
        r2 = jnp.where(last_r, zero, r + 1)
        pp2 = jnp.where(last_r, 1 - pp, pp)

        pltpu.make_async_copy(
            hn_hbm.at[pl.ds(0, K)], rows2.at[pl.ds(p * K, K)], semg.at[p]).wait()
        for g in range(K // 16):
            sv = src_sup[irow, pl.ds(g * 16, 16)]
            dv = dst_sup[irow, pl.ds(g * 16, 16)]
            av = plsc.load_gather(aa_v, [dv])
            bv = plsc.load_gather(bb_v, [sv])
            x = av + bv
            t = jnp.exp(-2.0 * jnp.abs(x))
            ta = (1.0 - t) / (1.0 + t)
            w_v[pl.ds(g * 16, 16)] = jnp.where(x >= 0.0, ta, -ta)

        @pl.when(jnp.logical_and(last_r, su < NSUP - 1))
        def _():
            pltpu.sync_copy(src4_hbm.at[wid, su + 1],
                            src_sup.at[pl.ds((1 - pp) * SUPC, SUPC)])
            pltpu.sync_copy(dst4_hbm.at[wid, su + 1],
                            dst_sup.at[pl.ds((1 - pp) * SUPC, SUPC)])

        for rr in range(16):
            wb = plsc.load_gather(w_v, [jnp.full((16,), rr, _i32)])
            for j in range(DIM // 16):
                rows2[p * K + rr, pl.ds(j * 16, 16)] = (
                    rows2[p * K + rr, pl.ds(j * 16, 16)] * wb)

        @pl.when(ci < NCHUNK - 1)
        def _():
            @pl.when(ci > 0)
            def _():
                pltpu.make_async_copy(
                    rows2.at[pl.ds(q * K, K)], macc.at[pl.ds(0, K)],
                    sems.at[q]).wait()
            pltpu.async_copy(hn_hbm.at[src_sup.at[pp2 * SUPC + r2]],
                             rows2.at[pl.ds(q * K, K)], semg.at[q])

        for rr in range(16, K):
            wb = plsc.load_gather(w_v, [jnp.full((16,), rr, _i32)])
            for j in range(DIM // 16):
                rows2[p * K + rr, pl.ds(j * 16, 16)] = (
                    rows2[p * K + rr, pl.ds(j * 16, 16)] * wb)
        pltpu.async_copy(rows2.at[pl.ds(p * K, K)], macc.at[dst_sup.at[irow]],
                         sems.at[p], add=True)

        return (q, su2, r2, pp2)

    lax.fori_loop(0, NCHUNK, chunk_body, (zero, zero, zero, zero))
    pltpu.make_async_copy(rows2.at[pl.ds(0, K)], macc.at[pl.ds(0, K)],
                          sems.at[0]).wait()
    pltpu.make_async_copy(rows2.at[pl.ds(K, K)], macc.at[pl.ds(0, K)],
                          sems.at[1]).wait()
    plsc.subcore_barrier()

    def rstripe(i, carry):
        g = s + NS * i
        pltpu.sync_copy(macc.at[pl.ds(g * 8, 8)],
                        mparts_hbm.at[c, pl.ds(g * 8, 8)])
        return carry

    lax.fori_loop(0, my_ng, rstripe, 0)


def _tc_smalls_body(h0, degp, m2d, diagt, m2p, prot, vwd, vwp, vb2, ehr,
                    selwT, selb2, ctxwT, ctxb2, wmat0, gb0,
                    attn_o, normc_o, hn0_o, aa0_o, bb0_o):
    ones32 = jnp.ones((NW, 1), _f32)
    deg = lax.dot_general(degp[...], ones32, (((0,), (0,)), ((), ())))
    normc = lax.rsqrt(jnp.maximum(deg, 1.0))
    normc_o[...] = normc
    hn0_o[...] = h0[...] * normc
    ab = h0[...] @ wmat0[...]
    aa0_o[...] = ab[:, 0:1] + gb0[...]
    bb0_o[...] = ab[:, 1:2]

    dv = m2d[...] @ diagt[...]
    pv = m2p[...] @ prot[...]
    med_rec = dv @ vwd[...] + pv @ vwp[...] + vb2[...]
    aug = ehr[...] @ med_rec
    sel = jnp.tanh(med_rec @ selwT[...] + selb2[...])
    context = med_rec + sel * aug
    attn_o[...] = jnp.tanh(context @ ctxwT[...] + ctxb2[...])


def _tc_update_body(h, m0, m1, normc, attn_full, wmat, gb,
                    h1_o, hn1_o, aa1_o, bb1_o):
    m = (m0[...] + m1[...]) * normc[...]
    hnew = jnp.maximum(EPS * h[...] + attn_full[...] * m, 0.0)
    h1_o[...] = hnew
    hn1_o[...] = hnew * normc[...]
    ab = hnew @ wmat[...]
    aa1_o[...] = ab[:, 0:1] + gb[...]
    bb1_o[...] = ab[:, 1:2]


def _tc_final_body(h, m0, m1, normc, attn_full, avgp, out_o):
    m = (m0[...] + m1[...]) * normc[...]
    h2 = jnp.maximum(EPS * h[...] + attn_full[...] * m, 0.0)
    mol = h2.reshape(NMED, MOL_SIZE, DIM).sum(axis=1)
    out_o[...] = avgp[...] @ mol


def _shape(s, d=_f32):
    return jax.ShapeDtypeStruct(s, d)


def kernel(fingerprints, edge_index, diag_table, pro_table, med2diag, med2pro,
           ehradj_idx, embed_table, gate_w0, gate_b0, gate_w1, gate_b1,
           ctx_w, ctx_b, viewcat_w, viewcat_b, sel_w, sel_b, avg_projection):
    src = edge_index[0]
    dst = edge_index[1]
    src4 = src.reshape(NW, NSUP, SUPC, K)
    dst4 = dst.reshape(NW, NSUP, SUPC, K)

    h0, degp = _sc_prep(fingerprints.astype(_i32), dst, embed_table)
    degp = degp.reshape(NW, N_NODES)

    vwT = viewcat_w.T
    wmat0 = gate_w0.reshape(2, DIM).T
    wmat1 = gate_w1.reshape(2, DIM).T
    attn_s, normc, hn0, aa0, bb0 = pl.pallas_call(
        _tc_smalls_body,
        out_shape=(
            _shape((NMED, DIM)), _shape((N_NODES, 1)), _shape((N_NODES, DIM)),
            _shape((N_NODES, 1)), _shape((N_NODES, 1)),
        ),
    )(h0, degp, med2diag, diag_table, med2pro, pro_table,
      vwT[:DIM], vwT[DIM:], viewcat_b.reshape(1, DIM), ehradj_idx,
      sel_w.T, sel_b.reshape(1, DIM), ctx_w.T, ctx_b.reshape(1, DIM),
      wmat0, gate_b0.reshape(1, 1))

    attn_full = jnp.repeat(attn_s, MOL_SIZE, axis=0)

    m = _sc_edge(src4, dst4, hn0,
                 aa0.reshape(N_NODES), bb0.reshape(N_NODES))
    h1, hn1, aa1, bb1 = pl.pallas_call(
        _tc_update_body,
        out_shape=(
            _shape((N_NODES, DIM)), _shape((N_NODES, DIM)),
            _shape((N_NODES, 1)), _shape((N_NODES, 1)),
        ),
    )(h0, m[0], m[1], normc, attn_full, wmat1, gate_b1.reshape(1, 1))

    m2 = _sc_edge(src4, dst4, hn1,
                  aa1.reshape(N_NODES), bb1.reshape(N_NODES))
    out = pl.pallas_call(
        _tc_final_body,
        out_shape=_shape((NMED, DIM)),
    )(h1, m2[0], m2[1], normc, attn_full, avg_projection)
    return out


# speedup vs baseline: 1.0567x; 1.0567x over previous
import functools

import jax
import jax.numpy as jnp
from jax import lax
from jax.experimental import pallas as pl
from jax.experimental.pallas import tpu as pltpu
from jax.experimental.pallas import tpu_sc as plsc

N_NODES = 10000
N_EDGES = 320000
DIM = 128
NMED = 200
MOL_SIZE = 50
EPS = 0.3

NC = 2
NS = 16
NW = NC * NS
EPW = N_EDGES // NW
K = 80
NSUP = 25
SUPC = 5
NCHUNK = NSUP * SUPC
NGROUP = N_NODES // 16

_mesh = plsc.VectorSubcoreMesh(
    core_axis_name="c", subcore_axis_name="s", num_cores=NC, num_subcores=NS)

_f32 = jnp.float32
_i32 = jnp.int32


@functools.partial(
    pl.kernel,
    out_type=(
        jax.ShapeDtypeStruct((N_NODES, DIM), _f32),
        jax.ShapeDtypeStruct((NW * N_NODES,), _f32),
    ),
    mesh=_mesh,
    scratch_types=[
        pltpu.VMEM((16,), _i32),
        pltpu.VMEM((N_NODES,), _i32),
        pltpu.VMEM((N_NODES,), _f32),
        pltpu.VMEM((16, DIM), _f32),
        pltpu.SemaphoreType.DMA,
    ],
    compiler_params=pltpu.CompilerParams(needs_layout_passes=False),
)
def _sc_prep(fp_hbm, dst_hbm, tbl_hbm, h0_hbm, degp_hbm,
             fp_v, dstbuf_v, hist_v, rows_v, sem):
    c = lax.axis_index("c")
    s = lax.axis_index("s")
    wid = s * NC + c

    n_groups = jnp.where(wid < (NGROUP % NW) + ((NGROUP % NW == 0) * NW),
                         NGROUP // NW + 1, NGROUP // NW)

    def emb_body(i, carry):
        g = wid + NW * i
        pltpu.sync_copy(fp_hbm.at[pl.ds(g * 16, 16)], fp_v)
        pltpu.async_copy(tbl_hbm.at[fp_v], rows_v, sem).wait()
        pltpu.sync_copy(rows_v, h0_hbm.at[pl.ds(g * 16, 16)])
        return carry

    lax.fori_loop(0, n_groups, emb_body, 0)

    def zero_body(i, carry):
        hist_v[pl.ds(i * 16, 16)] = jnp.zeros((16,), _f32)
        return carry

    lax.fori_loop(0, NGROUP, zero_body, 0)
    pltpu.sync_copy(dst_hbm.at[pl.ds(wid * EPW, EPW)], dstbuf_v)
    ones = jnp.ones((16,), _f32)

    def hist_body(i, carry):
        dv = dstbuf_v[pl.ds(i * 16, 16)]
        plsc.addupdate_scatter(hist_v, [dv], ones)
        return carry

    lax.fori_loop(0, EPW // 16, hist_body, 0)
    pltpu.sync_copy(hist_v, degp_hbm.at[pl.ds(wid * N_NODES, N_NODES)])


@functools.partial(
    pl.kernel,
    out_type=jax.ShapeDtypeStruct((NC, N_NODES, DIM), _f32),
    mesh=_mesh,
    scratch_types=[
        pltpu.VMEM((N_NODES,), _f32),
        pltpu.VMEM((N_NODES,), _f32),
        pltpu.VMEM((2 * SUPC, K), _i32),
        pltpu.VMEM((2 * SUPC, K), _i32),
        pltpu.VMEM((K,), _f32),
        pltpu.VMEM((2 * K, DIM), _f32),
        pltpu.VMEM_SHARED((N_NODES, DIM), _f32),
        pltpu.SemaphoreType.DMA((2,)),
        pltpu.SemaphoreType.DMA((2,)),
    ],
    compiler_params=pltpu.CompilerParams(needs_layout_passes=False),
)
def _sc_edge(src4_hbm, dst4_hbm, hn_hbm, aa_hbm, bb_hbm, mparts_hbm,
             aa_v, bb_v, src_sup, dst_sup, w_v, rows2, macc, semg, sems):
    c = lax.axis_index("c")
    s = lax.axis_index("s")
    wid = s * NC + c

    for i in range(8):
        for j in range(DIM // 16):
            rows2[i, pl.ds(j * 16, 16)] = jnp.zeros((16,), _f32)
    ngrp8 = N_NODES // 8
    my_ng = jnp.where(s < ngrp8 % NS, ngrp8 // NS + 1, ngrp8 // NS)

    def zstripe(i, carry):
        g = s + NS * i
        pltpu.sync_copy(rows2.at[pl.ds(0, 8)], macc.at[pl.ds(g * 8, 8)])
        return carry

    lax.fori_loop(0, my_ng, zstripe, 0)

    pltpu.sync_copy(aa_hbm, aa_v)
    pltpu.sync_copy(bb_hbm, bb_v)
    pltpu.sync_copy(src4_hbm.at[wid, 0], src_sup.at[pl.ds(0, SUPC)])
    pltpu.sync_copy(dst4_hbm.at[wid, 0], dst_sup.at[pl.ds(0, SUPC)])
    plsc.subcore_barrier()
    pltpu.async_copy(hn_hbm.at[src_sup.at[0]], rows2.at[pl.ds(0, K)], semg.at[0])

    zero = jnp.int32(0)

    def chunk_body(ci, carry):
        p, su, r, pp = carry
        irow = pp * SUPC + r
        last_r = r == SUPC - 1
        q = 1 - p
        su2 = jnp.where(last_r, su + 1, su)
        r2 = jnp.where(last_r, zero, r + 1)
        pp2 = jnp.where(last_r, 1 - pp, pp)

        pltpu.make_async_copy(
            hn_hbm.at[pl.ds(0, K)], rows2.at[pl.ds(p * K, K)], semg.at[p]).wait()
        for g in range(K // 16):
            sv = src_sup[irow, pl.ds(g * 16, 16)]
            dv = dst_sup[irow, pl.ds(g * 16, 16)]
            av = plsc.load_gather(aa_v, [dv])
            bv = plsc.load_gather(bb_v, [sv])
            x = av + bv
            t = jnp.exp(-2.0 * jnp.abs(x))
            ta = (1.0 - t) / (1.0 + t)
            w_v[pl.ds(g * 16, 16)] = jnp.where(x >= 0.0, ta, -ta)

        @pl.when(jnp.logical_and(last_r, su < NSUP - 1))
        def _():
            pltpu.sync_copy(src4_hbm.at[wid, su + 1],
                            src_sup.at[pl.ds((1 - pp) * SUPC, SUPC)])
            pltpu.sync_copy(dst4_hbm.at[wid, su + 1],
                            dst_sup.at[pl.ds((1 - pp) * SUPC, SUPC)])

        @pl.when(ci < NCHUNK - 1)
        def _():
            @pl.when(ci > 0)
            def _():
                pltpu.make_async_copy(
                    rows2.at[pl.ds(q * K, K)], macc.at[pl.ds(0, K)],
                    sems.at[q]).wait()
            pltpu.async_copy(hn_hbm.at[src_sup.at[pp2 * SUPC + r2]],
                             rows2.at[pl.ds(q * K, K)], semg.at[q])

        for rr in range(K):
            wb = plsc.load_gather(w_v, [jnp.full((16,), rr, _i32)])
            for j in range(DIM // 16):
                rows2[p * K + rr, pl.ds(j * 16, 16)] = (
                    rows2[p * K + rr, pl.ds(j * 16, 16)] * wb)
        pltpu.async_copy(rows2.at[pl.ds(p * K, K)], macc.at[dst_sup.at[irow]],
                         sems.at[p], add=True)

        return (q, su2, r2, pp2)

    lax.fori_loop(0, NCHUNK, chunk_body, (zero, zero, zero, zero))
    pltpu.make_async_copy(rows2.at[pl.ds(0, K)], macc.at[pl.ds(0, K)],
                          sems.at[0]).wait()
    pltpu.make_async_copy(rows2.at[pl.ds(K, K)], macc.at[pl.ds(0, K)],
                          sems.at[1]).wait()
    plsc.subcore_barrier()

    def rstripe(i, carry):
        g = s + NS * i
        pltpu.sync_copy(macc.at[pl.ds(g * 8, 8)],
                        mparts_hbm.at[c, pl.ds(g * 8, 8)])
        return carry

    lax.fori_loop(0, my_ng, rstripe, 0)


def _tc_smalls_body(h0, degp, m2d, diagt, m2p, prot, vwd, vwp, vb2, ehr,
                    selwT, selb2, ctxwT, ctxb2, wmat0, gb0,
                    attn_o, normc_o, hn0_o, aa0_o, bb0_o):
    ones32 = jnp.ones((NW, 1), _f32)
    deg = lax.dot_general(degp[...], ones32, (((0,), (0,)), ((), ())))
    normc = lax.rsqrt(jnp.maximum(deg, 1.0))
    normc_o[...] = normc
    hn0_o[...] = h0[...] * normc
    ab = h0[...] @ wmat0[...]
    aa0_o[...] = ab[:, 0:1] + gb0[...]
    bb0_o[...] = ab[:, 1:2]

    dv = m2d[...] @ diagt[...]
    pv = m2p[...] @ prot[...]
    med_rec = dv @ vwd[...] + pv @ vwp[...] + vb2[...]
    aug = ehr[...] @ med_rec
    sel = jnp.tanh(med_rec @ selwT[...] + selb2[...])
    context = med_rec + sel * aug
    attn_o[...] = jnp.tanh(context @ ctxwT[...] + ctxb2[...])


def _tc_update_body(h, m0, m1, normc, attn_full, wmat, gb,
                    h1_o, hn1_o, aa1_o, bb1_o):
    m = (m0[...] + m1[...]) * normc[...]
    hnew = jnp.maximum(EPS * h[...] + attn_full[...] * m, 0.0)
    h1_o[...] = hnew
    hn1_o[...] = hnew * normc[...]
    ab = hnew @ wmat[...]
    aa1_o[...] = ab[:, 0:1] + gb[...]
    bb1_o[...] = ab[:, 1:2]


def _tc_final_body(h, m0, m1, normc, attn_full, avgp, out_o):
    m = (m0[...] + m1[...]) * normc[...]
    h2 = jnp.maximum(EPS * h[...] + attn_full[...] * m, 0.0)
    mol = h2.reshape(NMED, MOL_SIZE, DIM).sum(axis=1)
    out_o[...] = avgp[...] @ mol


def _shape(s, d=_f32):
    return jax.ShapeDtypeStruct(s, d)


def kernel(fingerprints, edge_index, diag_table, pro_table, med2diag, med2pro,
           ehradj_idx, embed_table, gate_w0, gate_b0, gate_w1, gate_b1,
           ctx_w, ctx_b, viewcat_w, viewcat_b, sel_w, sel_b, avg_projection):
    src = edge_index[0]
    dst = edge_index[1]
    src4 = src.reshape(NW, NSUP, SUPC, K)
    dst4 = dst.reshape(NW, NSUP, SUPC, K)

    h0, degp = _sc_prep(fingerprints.astype(_i32), dst, embed_table)
    degp = degp.reshape(NW, N_NODES)

    vwT = viewcat_w.T
    wmat0 = gate_w0.reshape(2, DIM).T
    wmat1 = gate_w1.reshape(2, DIM).T
    attn_s, normc, hn0, aa0, bb0 = pl.pallas_call(
        _tc_smalls_body,
        out_shape=(
            _shape((NMED, DIM)), _shape((N_NODES, 1)), _shape((N_NODES, DIM)),
            _shape((N_NODES, 1)), _shape((N_NODES, 1)),
        ),
    )(h0, degp, med2diag, diag_table, med2pro, pro_table,
      vwT[:DIM], vwT[DIM:], viewcat_b.reshape(1, DIM), ehradj_idx,
      sel_w.T, sel_b.reshape(1, DIM), ctx_w.T, ctx_b.reshape(1, DIM),
      wmat0, gate_b0.reshape(1, 1))

    attn_full = jnp.repeat(attn_s, MOL_SIZE, axis=0)

    m = _sc_edge(src4, dst4, hn0,
                 aa0.reshape(N_NODES), bb0.reshape(N_NODES))
    h1, hn1, aa1, bb1 = pl.pallas_call(
        _tc_update_body,
        out_shape=(
            _shape((N_NODES, DIM)), _shape((N_NODES, DIM)),
            _shape((N_NODES, 1)), _shape((N_NODES, 1)),
        ),
    )(h0, m[0], m[1], normc, attn_full, wmat1, gate_b1.reshape(1, 1))

    m2 = _sc_edge(src4, dst4, hn1,
                  aa1.reshape(N_NODES), bb1.reshape(N_NODES))
    out = pl.pallas_call(
        _tc_final_body,
        out_shape=_shape((NMED, DIM)),
    )(h1, m2[0], m2[1], normc, attn_full, avg_projection)
    return out
